# baseline (speedup 1.0000x reference)
import functools

import jax
import jax.numpy as jnp
from jax import lax
from jax.experimental import pallas as pl
from jax.experimental.pallas import tpu as pltpu
from jax.experimental.pallas import tpu_sc as plsc

PAD = 0
D = 64
LANES = 16
NC, NS = 2, 16
NW = NC * NS
IW = 128
C = 512
KR = C // IW


def _scalar_min16(vec):
    m = vec[0]
    for u in range(1, LANES):
        m = jnp.minimum(m, vec[u])
    return m


@functools.lru_cache(maxsize=None)
def _emb_call(B):
    per_w = B // NW
    nch = per_w // C
    assert B % NW == 0 and per_w % C == 0
    assert nch >= 4 and nch % 2 == 0

    mesh = plsc.VectorSubcoreMesh(core_axis_name="c", subcore_axis_name="s",
                                  num_cores=NC, num_subcores=NS)

    @functools.partial(
        pl.kernel,
        out_type=jax.ShapeDtypeStruct((B, D), jnp.float32),
        mesh=mesh,
        compiler_params=pltpu.CompilerParams(use_tc_tiling_on_sc=False),
        scratch_types=[
            pltpu.VMEM((2, C), jnp.int32),
            pltpu.VMEM((2, C, D), jnp.float32),
            pltpu.SemaphoreType.DMA,
            pltpu.SemaphoreType.DMA,
            pltpu.SemaphoreType.DMA,
            pltpu.SemaphoreType.DMA,
        ],
    )
    def run(ids_hbm, table_hbm, out_hbm, idx_v, rows_v, sg0, sg1, sw0, sw1):
        sg = (sg0, sg1)
        sw = (sw0, sw1)
        wid = lax.axis_index("s") * NC + lax.axis_index("c")
        base0 = wid * per_w

        def fire_chunk(j, b):
            pltpu.sync_copy(ids_hbm.at[pl.ds(base0 + j * C, C)], idx_v.at[b])
            for r in range(KR):
                pltpu.async_copy(table_hbm.at[idx_v.at[b, pl.ds(r * IW, IW)]],
                                 rows_v.at[b, pl.ds(r * IW, IW)], sg[b])

        def wait_gather(b):
            for r in range(KR):
                pltpu.make_async_copy(
                    table_hbm.at[idx_v.at[b, pl.ds(r * IW, IW)]],
                    rows_v.at[b, pl.ds(r * IW, IW)], sg[b]).wait()

        def fire_write(j, b):
            pltpu.async_copy(rows_v.at[b],
                             out_hbm.at[pl.ds(base0 + j * C, C)], sw[b])

        def wait_write(j, b):
            pltpu.make_async_copy(rows_v.at[b],
                                  out_hbm.at[pl.ds(base0 + j * C, C)],
                                  sw[b]).wait()

        def fix(b):
            def gmin(g, acc):
                return jnp.minimum(acc, idx_v[b, pl.ds(g * LANES, LANES)])

            acc = lax.fori_loop(0, C // LANES, gmin,
                                jnp.full((LANES,), jnp.iinfo(jnp.int32).max,
                                         jnp.int32))

            @pl.when(_scalar_min16(acc) == PAD)
            def _():
                def gfix(g, carry):
                    ivec = idx_v[b, pl.ds(g * LANES, LANES)]

                    @pl.when(_scalar_min16(ivec) == PAD)
                    def _():
                        for u in range(LANES):
                            scale = jnp.where(ivec[u] == PAD, 0.0, 1.0)
                            for cc in range(D // LANES):
                                sl = pl.ds(cc * LANES, LANES)
                                rows_v[b, g * LANES + u, sl] = (
                                    rows_v[b, g * LANES + u, sl] * scale)

                    return carry

                lax.fori_loop(0, C // LANES, gfix, 0)

        fire_chunk(0, 0)
        wait_gather(0)
        fix(0)
        fire_chunk(1, 1)
        fire_write(0, 0)

        def main_pair(i, carry):
            j = 1 + 2 * i
            for b, dj in ((1, 0), (0, 1)):
                jj = j + dj
                wait_gather(b)
                fix(b)
                fire_write(jj, b)
                wait_write(jj - 1, 1 - b)
                fire_chunk(jj + 1, 1 - b)
            return carry

        lax.fori_loop(0, (nch - 2) // 2, main_pair, 0)

        wait_gather(1)
        fix(1)
        fire_write(nch - 1, 1)
        wait_write(nch - 2, 0)
        wait_write(nch - 1, 1)

    return run


def kernel(ids, table):
    shp = ids.shape
    B = ids.size
    out = _emb_call(B)(ids.reshape(B).astype(jnp.int32), table)
    return out.reshape(*shp, D)


# speedup vs baseline: 1.0160x; 1.0160x over previous
import functools

import jax
import jax.numpy as jnp
from jax import lax
from jax.experimental import pallas as pl
from jax.experimental.pallas import tpu as pltpu
from jax.experimental.pallas import tpu_sc as plsc

PAD = 0
D = 64
LANES = 16
NC, NS = 2, 16
NW = NC * NS
C = 512


def _scalar_min16(vec):
    m = vec[0]
    for u in range(1, LANES):
        m = jnp.minimum(m, vec[u])
    return m


@functools.lru_cache(maxsize=None)
def _emb_call(B):
    per_w = B // NW
    nch = per_w // C
    assert B % NW == 0 and per_w % C == 0
    assert nch >= 4 and nch % 2 == 0

    mesh = plsc.VectorSubcoreMesh(core_axis_name="c", subcore_axis_name="s",
                                  num_cores=NC, num_subcores=NS)

    @functools.partial(
        pl.kernel,
        out_type=jax.ShapeDtypeStruct((B, D), jnp.float32),
        mesh=mesh,
        compiler_params=pltpu.CompilerParams(use_tc_tiling_on_sc=False),
        scratch_types=[
            pltpu.VMEM((per_w,), jnp.int32),
            pltpu.VMEM((2, C, D), jnp.float32),
            pltpu.SemaphoreType.DMA,
            pltpu.SemaphoreType.DMA,
            pltpu.SemaphoreType.DMA,
            pltpu.SemaphoreType.DMA,
        ],
    )
    def run(ids_hbm, table_hbm, out_hbm, idx_v, rows_v, sg0, sg1, sw0, sw1):
        sg = (sg0, sg1)
        sw = (sw0, sw1)
        wid = lax.axis_index("s") * NC + lax.axis_index("c")
        base0 = wid * per_w

        pltpu.sync_copy(ids_hbm.at[pl.ds(base0, per_w)], idx_v)

        def fire_chunk(j, b):
            pltpu.async_copy(table_hbm.at[idx_v.at[pl.ds(j * C, C)]],
                             rows_v.at[b], sg[b])

        def wait_gather(j, b):
            pltpu.make_async_copy(table_hbm.at[idx_v.at[pl.ds(j * C, C)]],
                                  rows_v.at[b], sg[b]).wait()

        def fire_write(j, b):
            pltpu.async_copy(rows_v.at[b],
                             out_hbm.at[pl.ds(base0 + j * C, C)], sw[b])

        def wait_write(j, b):
            pltpu.make_async_copy(rows_v.at[b],
                                  out_hbm.at[pl.ds(base0 + j * C, C)],
                                  sw[b]).wait()

        def fix(j, b):
            def gmin(g, acc):
                return jnp.minimum(acc, idx_v[pl.ds(j * C + g * LANES, LANES)])

            acc = lax.fori_loop(0, C // LANES, gmin,
                                jnp.full((LANES,), jnp.iinfo(jnp.int32).max,
                                         jnp.int32))

            @pl.when(_scalar_min16(acc) == PAD)
            def _():
                def gfix(g, carry):
                    ivec = idx_v[pl.ds(j * C + g * LANES, LANES)]

                    @pl.when(_scalar_min16(ivec) == PAD)
                    def _():
                        for u in range(LANES):
                            scale = jnp.where(ivec[u] == PAD, 0.0, 1.0)
                            for cc in range(D // LANES):
                                sl = pl.ds(cc * LANES, LANES)
                                rows_v[b, g * LANES + u, sl] = (
                                    rows_v[b, g * LANES + u, sl] * scale)

                    return carry

                lax.fori_loop(0, C // LANES, gfix, 0)

        fire_chunk(0, 0)
        wait_gather(0, 0)
        fix(0, 0)
        fire_chunk(1, 1)
        fire_write(0, 0)

        def main_pair(i, carry):
            j = 1 + 2 * i
            for b, dj in ((1, 0), (0, 1)):
                jj = j + dj
                wait_gather(jj, b)
                fix(jj, b)
                fire_write(jj, b)
                wait_write(jj - 1, 1 - b)
                fire_chunk(jj + 1, 1 - b)
            return carry

        lax.fori_loop(0, (nch - 2) // 2, main_pair, 0)

        wait_gather(nch - 1, 1)
        fix(nch - 1, 1)
        fire_write(nch - 1, 1)
        wait_write(nch - 2, 0)
        wait_write(nch - 1, 1)

    return run


def kernel(ids, table):
    shp = ids.shape
    B = ids.size
    out = _emb_call(B)(ids.reshape(B).astype(jnp.int32), table)
    return out.reshape(*shp, D)


# speedup vs baseline: 1.3490x; 1.3277x over previous
import functools

import jax
import jax.numpy as jnp
from jax import lax
from jax.experimental import pallas as pl
from jax.experimental.pallas import tpu as pltpu
from jax.experimental.pallas import tpu_sc as plsc

PAD = 0
D = 64
LANES = 16
NC, NS = 2, 16
NW = NC * NS
C = 512


def _scalar_min16(vec):
    m = vec[0]
    for u in range(1, LANES):
        m = jnp.minimum(m, vec[u])
    return m


@functools.lru_cache(maxsize=None)
def _emb_call(B):
    per_w = B // NW
    nch = per_w // C
    assert B % NW == 0 and per_w % C == 0
    assert nch >= 4 and nch % 2 == 0

    mesh = plsc.VectorSubcoreMesh(core_axis_name="c", subcore_axis_name="s",
                                  num_cores=NC, num_subcores=NS)

    @functools.partial(
        pl.kernel,
        out_type=jax.ShapeDtypeStruct((B, 2 * D), jnp.float32),
        mesh=mesh,
        compiler_params=pltpu.CompilerParams(use_tc_tiling_on_sc=False),
        scratch_types=[
            pltpu.VMEM((per_w,), jnp.int32),
            pltpu.VMEM((2, C, D), jnp.float32),
            pltpu.SemaphoreType.DMA,
            pltpu.SemaphoreType.DMA,
            pltpu.SemaphoreType.DMA,
            pltpu.SemaphoreType.DMA,
        ],
    )
    def run(ids_hbm, table_hbm, out_hbm, idx_v, rows_v, sg0, sg1, sw0, sw1):
        sg = (sg0, sg1)
        sw = (sw0, sw1)
        wid = lax.axis_index("s") * NC + lax.axis_index("c")
        base0 = wid * per_w

        pltpu.sync_copy(ids_hbm.at[pl.ds(base0, per_w)], idx_v)

        def fire_chunk(j, b):
            pltpu.async_copy(table_hbm.at[idx_v.at[pl.ds(j * C, C)]],
                             rows_v.at[b], sg[b])

        def wait_gather(j, b):
            pltpu.make_async_copy(table_hbm.at[idx_v.at[pl.ds(j * C, C)]],
                                  rows_v.at[b], sg[b]).wait()

        def fire_write(j, b):
            pltpu.async_copy(rows_v.at[b],
                             out_hbm.at[pl.ds(base0 + j * C, C), pl.ds(0, D)],
                             sw[b])

        def wait_write(j, b):
            pltpu.make_async_copy(
                rows_v.at[b],
                out_hbm.at[pl.ds(base0 + j * C, C), pl.ds(0, D)],
                sw[b]).wait()

        def fix(j, b):
            def gmin(g, acc):
                return jnp.minimum(acc, idx_v[pl.ds(j * C + g * LANES, LANES)])

            acc = lax.fori_loop(0, C // LANES, gmin,
                                jnp.full((LANES,), jnp.iinfo(jnp.int32).max,
                                         jnp.int32))

            @pl.when(_scalar_min16(acc) == PAD)
            def _():
                def gfix(g, carry):
                    ivec = idx_v[pl.ds(j * C + g * LANES, LANES)]

                    @pl.when(_scalar_min16(ivec) == PAD)
                    def _():
                        for u in range(LANES):
                            scale = jnp.where(ivec[u] == PAD, 0.0, 1.0)
                            for cc in range(D // LANES):
                                sl = pl.ds(cc * LANES, LANES)
                                rows_v[b, g * LANES + u, sl] = (
                                    rows_v[b, g * LANES + u, sl] * scale)

                    return carry

                lax.fori_loop(0, C // LANES, gfix, 0)

        fire_chunk(0, 0)
        wait_gather(0, 0)
        fix(0, 0)
        fire_chunk(1, 1)
        fire_write(0, 0)

        def main_pair(i, carry):
            j = 1 + 2 * i
            for b, dj in ((1, 0), (0, 1)):
                jj = j + dj
                wait_gather(jj, b)
                fix(jj, b)
                fire_write(jj, b)
                wait_write(jj - 1, 1 - b)
                fire_chunk(jj + 1, 1 - b)
            return carry

        lax.fori_loop(0, (nch - 2) // 2, main_pair, 0)

        wait_gather(nch - 1, 1)
        fix(nch - 1, 1)
        fire_write(nch - 1, 1)
        wait_write(nch - 2, 0)
        wait_write(nch - 1, 1)

    return run


def kernel(ids, table):
    shp = ids.shape
    B = ids.size
    out = _emb_call(B)(ids.reshape(B).astype(jnp.int32), table)
    return out.reshape(*shp, 2 * D)[..., :D]
